# Initial kernel scaffold; baseline (speedup 1.0000x reference)
import functools

import jax
import jax.numpy as jnp
from jax import lax
from jax.experimental import pallas as pl
from jax.experimental.pallas import tpu as pltpu
from jax.experimental.pallas import tpu_sc as plsc

N = 10000
E = 320000
D = 128
NP = 10240
NC, NS = 2, 16
NW = NC * NS
EPT = E // NW
C = 80
NCHUNK = EPT // C
RPT = NP // NS
CNTW = 16
BM = 1024


def _make_sc_agg(with_count):
  outs = [jax.ShapeDtypeStruct((NC, NP, D), jnp.float32)]
  scratch = [
      pltpu.VMEM((C,), jnp.int32),
      pltpu.VMEM((C,), jnp.int32),
      pltpu.VMEM((C, D), jnp.float32),
      pltpu.SemaphoreType.DMA,
  ]
  if with_count:
    outs.append(jax.ShapeDtypeStruct((NC, NP, CNTW), jnp.float32))
    scratch.append(pltpu.VMEM((C, CNTW), jnp.float32))
  scratch.append(pltpu.VMEM_SHARED((NP, D), jnp.float32))
  if with_count:
    scratch.append(pltpu.VMEM_SHARED((NP, CNTW), jnp.float32))

  def body(x_hbm, src_hbm, dst_hbm, *refs):
    if with_count:
      agg_hbm, cnt_hbm, sidx, didx, rows, sem, ones, acc_sh, cnt_sh = refs
    else:
      agg_hbm, sidx, didx, rows, sem, acc_sh = refs
    c = lax.axis_index("c")
    s = lax.axis_index("s")
    wid = s * NC + c

    zv = jnp.zeros((16,), jnp.float32)

    def zrow(i, _):
      for j in range(D // 16):
        rows[i, pl.ds(j * 16, 16)] = zv
      return 0

    lax.fori_loop(0, C, zrow, 0)
    for k in range(RPT // C):
      pltpu.sync_copy(rows, acc_sh.at[pl.ds(s * RPT + k * C, C)])
    if with_count:
      def zcnt(i, _):
        ones[i, :] = zv
        return 0

      lax.fori_loop(0, C, zcnt, 0)
      for k in range(RPT // C):
        pltpu.sync_copy(ones, cnt_sh.at[pl.ds(s * RPT + k * C, C)])
      ov = jnp.ones((16,), jnp.float32)

      def fill1(i, _):
        ones[i, :] = ov
        return 0

      lax.fori_loop(0, C, fill1, 0)
    plsc.subcore_barrier()

    def step(i, _):
      base = pl.multiple_of(wid * EPT + i * C, 8)
      pltpu.sync_copy(src_hbm.at[pl.ds(base, C)], sidx)
      pltpu.sync_copy(dst_hbm.at[pl.ds(base, C)], didx)
      pltpu.async_copy(x_hbm.at[sidx], rows, sem).wait()
      pltpu.sync_copy(rows, acc_sh.at[didx], add=True)
      if with_count:
        pltpu.sync_copy(ones, cnt_sh.at[didx], add=True)
      return 0

    lax.fori_loop(0, NCHUNK, step, 0)
    plsc.subcore_barrier()

    r0 = pl.multiple_of(s * RPT, 8)
    pltpu.sync_copy(acc_sh.at[pl.ds(r0, RPT)], agg_hbm.at[c, pl.ds(r0, RPT)])
    if with_count:
      pltpu.sync_copy(cnt_sh.at[pl.ds(r0, RPT)], cnt_hbm.at[c, pl.ds(r0, RPT)])

  return pl.kernel(
      body,
      out_type=tuple(outs) if with_count else outs[0],
      mesh=plsc.VectorSubcoreMesh(core_axis_name="c", subcore_axis_name="s"),
      scratch_types=scratch,
  )


_sc_agg_cnt = _make_sc_agg(True)
_sc_agg = _make_sc_agg(False)


def _elu(h):
  return jnp.where(h > 0, h, jnp.exp(jnp.minimum(h, 0.0)) - 1.0)


def _tc1_body(agg_ref, cnt_ref, x_ref, wl_ref, bl_ref, wr_ref, o_ref):
  cnt = jnp.maximum(cnt_ref[:, 0] + cnt_ref[:, 1], 1.0)
  mean = (agg_ref[0] + agg_ref[1]) * (1.0 / cnt)[:, None]
  h = (jnp.dot(mean, wl_ref[...], preferred_element_type=jnp.float32)
       + jnp.dot(x_ref[...], wr_ref[...], preferred_element_type=jnp.float32)
       + bl_ref[...][None, :])
  o_ref[...] = _elu(h)


def _tc2_body(agg_ref, cnt_ref, h_ref, wl_ref, bl_ref, wr_ref,
              wf1_ref, bf1_ref, wf2_ref, bf2_ref, o_ref):
  cnt = jnp.maximum(cnt_ref[:, 0] + cnt_ref[:, 1], 1.0)
  mean = (agg_ref[0] + agg_ref[1]) * (1.0 / cnt)[:, None]
  h = (jnp.dot(mean, wl_ref[...], preferred_element_type=jnp.float32)
       + jnp.dot(h_ref[...], wr_ref[...], preferred_element_type=jnp.float32)
       + bl_ref[...][None, :])
  h = _elu(h)
  g = jnp.maximum(
      jnp.dot(h, wf1_ref[...], preferred_element_type=jnp.float32)
      + bf1_ref[...][None, :], 0.0)
  o_ref[...] = jnp.sum(g * wf2_ref[...][None, :], axis=1) + bf2_ref[...]


def kernel(x, edge_index, W1l, b1l, W1r, W2l, b2l, W2r, Wf1, bf1, Wf2, bf2):
  src = edge_index[0]
  dst = edge_index[1]

  agg1, cntp = _sc_agg_cnt(x, src, dst)
  cnt2 = cntp[:, :N, 0].T

  grid = (pl.cdiv(N, BM),)
  h1 = pl.pallas_call(
      _tc1_body,
      grid=grid,
      in_specs=[
          pl.BlockSpec((2, BM, D), lambda i: (0, i, 0)),
          pl.BlockSpec((BM, 2), lambda i: (i, 0)),
          pl.BlockSpec((BM, D), lambda i: (i, 0)),
          pl.BlockSpec((D, D), lambda i: (0, 0)),
          pl.BlockSpec((D,), lambda i: (0,)),
          pl.BlockSpec((D, D), lambda i: (0, 0)),
      ],
      out_specs=pl.BlockSpec((BM, D), lambda i: (i, 0)),
      out_shape=jax.ShapeDtypeStruct((N, D), jnp.float32),
  )(agg1, cnt2, x, W1l.T, b1l, W1r.T)

  agg2 = _sc_agg(h1, src, dst)

  out = pl.pallas_call(
      _tc2_body,
      grid=grid,
      in_specs=[
          pl.BlockSpec((2, BM, D), lambda i: (0, i, 0)),
          pl.BlockSpec((BM, 2), lambda i: (i, 0)),
          pl.BlockSpec((BM, D), lambda i: (i, 0)),
          pl.BlockSpec((D, D), lambda i: (0, 0)),
          pl.BlockSpec((D,), lambda i: (0,)),
          pl.BlockSpec((D, D), lambda i: (0, 0)),
          pl.BlockSpec((D, D // 2), lambda i: (0, 0)),
          pl.BlockSpec((D // 2,), lambda i: (0,)),
          pl.BlockSpec((D // 2,), lambda i: (0,)),
          pl.BlockSpec((1,), lambda i: (0,)),
      ],
      out_specs=pl.BlockSpec((BM,), lambda i: (i,)),
      out_shape=jax.ShapeDtypeStruct((N,), jnp.float32),
  )(agg2, cnt2, h1, W2l.T, b2l, W2r.T, Wf1.T, bf1, Wf2[0], bf2)

  return out


# speedup vs baseline: 4.7984x; 4.7984x over previous
import jax
import jax.numpy as jnp
from jax import lax
from jax.experimental import pallas as pl
from jax.experimental.pallas import tpu as pltpu
from jax.experimental.pallas import tpu_sc as plsc

N = 10000
E = 320000
D = 128
NP = 10240
NC, NS = 2, 16
NW = NC * NS
EPT = E // NW
C = 80
NCHUNK = EPT // C
RPT = NP // NS
BM = 1024


def _fill_iden(iden, s, lane):
  nj = C // 16

  def mkid(t, _):
    k = t // nj
    j = t % nj
    iden[k, pl.ds(j * 16, 16)] = s * RPT + k * C + j * 16 + lane
    return 0

  lax.fori_loop(0, (RPT // C) * nj, mkid, 0)


def _fill_const(buf, val):
  v = jnp.full((16,), val, jnp.float32)

  def fill(i, _):
    for j in range(D // 16):
      buf[i, pl.ds(j * 16, 16)] = v
    return 0

  lax.fori_loop(0, C, fill, 0)


def _sc_agg_body(x_hbm, src_hbm, dst_hbm, agg_hbm, sidx, didx, rows, iden,
                 sem, acc_sh):
  c = lax.axis_index("c")
  s = lax.axis_index("s")
  wid = s * NC + c

  lane = lax.iota(jnp.int32, 16)
  _fill_iden(iden, s, lane)
  _fill_const(rows, 0.0)

  for k in range(RPT // C):
    pltpu.sync_copy(rows, acc_sh.at[iden.at[k]])
  plsc.subcore_barrier()

  def step(i, _):
    base = pl.multiple_of(wid * EPT + i * C, 8)
    pltpu.sync_copy(src_hbm.at[pl.ds(base, C)], sidx)
    pltpu.sync_copy(dst_hbm.at[pl.ds(base, C)], didx)
    pltpu.async_copy(x_hbm.at[sidx], rows, sem).wait()
    pltpu.sync_copy(rows, acc_sh.at[didx], add=True)
    return 0

  lax.fori_loop(0, NCHUNK, step, 0)
  plsc.subcore_barrier()

  for k in range(RPT // C):
    off = pl.multiple_of(s * RPT + k * C, 8)
    pltpu.sync_copy(acc_sh.at[iden.at[k]], rows)
    pltpu.sync_copy(rows, agg_hbm.at[c, pl.ds(off, C)])


_sc_agg = pl.kernel(
    _sc_agg_body,
    out_type=jax.ShapeDtypeStruct((NC, NP, D), jnp.float32),
    mesh=plsc.VectorSubcoreMesh(core_axis_name="c", subcore_axis_name="s"),
    scratch_types=[
        pltpu.VMEM((C,), jnp.int32),
        pltpu.VMEM((C,), jnp.int32),
        pltpu.VMEM((C, D), jnp.float32),
        pltpu.VMEM((RPT // C, C), jnp.int32),
        pltpu.SemaphoreType.DMA,
        pltpu.VMEM_SHARED((NP, D), jnp.float32),
    ],
)


def _sc_cnt_body(dst_hbm, cnt_hbm, didx, ones, zero, iden, cnt_sh):
  c = lax.axis_index("c")
  s = lax.axis_index("s")
  wid = s * NC + c

  lane = lax.iota(jnp.int32, 16)
  _fill_iden(iden, s, lane)
  _fill_const(ones, 1.0)
  _fill_const(zero, 0.0)

  for k in range(RPT // C):
    pltpu.sync_copy(zero, cnt_sh.at[iden.at[k]])
  plsc.subcore_barrier()

  def step(i, _):
    base = pl.multiple_of(wid * EPT + i * C, 8)
    pltpu.sync_copy(dst_hbm.at[pl.ds(base, C)], didx)
    pltpu.sync_copy(ones, cnt_sh.at[didx], add=True)
    return 0

  lax.fori_loop(0, NCHUNK, step, 0)
  plsc.subcore_barrier()

  for k in range(RPT // C):
    off = pl.multiple_of(s * RPT + k * C, 8)
    pltpu.sync_copy(cnt_sh.at[iden.at[k]], zero)
    pltpu.sync_copy(zero, cnt_hbm.at[c, pl.ds(off, C)])


_sc_cnt = pl.kernel(
    _sc_cnt_body,
    out_type=jax.ShapeDtypeStruct((NC, NP, D), jnp.float32),
    mesh=plsc.VectorSubcoreMesh(core_axis_name="c", subcore_axis_name="s"),
    scratch_types=[
        pltpu.VMEM((C,), jnp.int32),
        pltpu.VMEM((C, D), jnp.float32),
        pltpu.VMEM((C, D), jnp.float32),
        pltpu.VMEM((RPT // C, C), jnp.int32),
        pltpu.VMEM_SHARED((NP, D), jnp.float32),
    ],
)


def _elu(h):
  return jnp.where(h > 0, h, jnp.exp(jnp.minimum(h, 0.0)) - 1.0)


def _tc1_body(agg_ref, cnt_ref, x_ref, wl_ref, bl_ref, wr_ref, o_ref):
  cnt = jnp.maximum(cnt_ref[:, 0] + cnt_ref[:, 1], 1.0)
  mean = (agg_ref[0] + agg_ref[1]) * (1.0 / cnt)[:, None]
  h = (jnp.dot(mean, wl_ref[...], preferred_element_type=jnp.float32)
       + jnp.dot(x_ref[...], wr_ref[...], preferred_element_type=jnp.float32)
       + bl_ref[...][None, :])
  o_ref[...] = _elu(h)


def _tc2_body(agg_ref, cnt_ref, h_ref, wl_ref, bl_ref, wr_ref,
              wf1_ref, bf1_ref, wf2_ref, bf2_ref, o_ref):
  cnt = jnp.maximum(cnt_ref[:, 0] + cnt_ref[:, 1], 1.0)
  mean = (agg_ref[0] + agg_ref[1]) * (1.0 / cnt)[:, None]
  h = (jnp.dot(mean, wl_ref[...], preferred_element_type=jnp.float32)
       + jnp.dot(h_ref[...], wr_ref[...], preferred_element_type=jnp.float32)
       + bl_ref[...][None, :])
  h = _elu(h)
  g = jnp.maximum(
      jnp.dot(h, wf1_ref[...], preferred_element_type=jnp.float32)
      + bf1_ref[...][None, :], 0.0)
  o_ref[...] = jnp.sum(g * wf2_ref[...][None, :], axis=1) + bf2_ref[...]


def kernel(x, edge_index, W1l, b1l, W1r, W2l, b2l, W2r, Wf1, bf1, Wf2, bf2):
  src = edge_index[0]
  dst = edge_index[1]

  cntp = _sc_cnt(dst)
  agg1 = _sc_agg(x, src, dst)
  cnt2 = cntp[:, :N, 0].T

  grid = (pl.cdiv(N, BM),)
  h1 = pl.pallas_call(
      _tc1_body,
      grid=grid,
      in_specs=[
          pl.BlockSpec((2, BM, D), lambda i: (0, i, 0)),
          pl.BlockSpec((BM, 2), lambda i: (i, 0)),
          pl.BlockSpec((BM, D), lambda i: (i, 0)),
          pl.BlockSpec((D, D), lambda i: (0, 0)),
          pl.BlockSpec((D,), lambda i: (0,)),
          pl.BlockSpec((D, D), lambda i: (0, 0)),
      ],
      out_specs=pl.BlockSpec((BM, D), lambda i: (i, 0)),
      out_shape=jax.ShapeDtypeStruct((N, D), jnp.float32),
  )(agg1, cnt2, x, W1l.T, b1l, W1r.T)

  agg2 = _sc_agg(h1, src, dst)

  out = pl.pallas_call(
      _tc2_body,
      grid=grid,
      in_specs=[
          pl.BlockSpec((2, BM, D), lambda i: (0, i, 0)),
          pl.BlockSpec((BM, 2), lambda i: (i, 0)),
          pl.BlockSpec((BM, D), lambda i: (i, 0)),
          pl.BlockSpec((D, D), lambda i: (0, 0)),
          pl.BlockSpec((D,), lambda i: (0,)),
          pl.BlockSpec((D, D), lambda i: (0, 0)),
          pl.BlockSpec((D, D // 2), lambda i: (0, 0)),
          pl.BlockSpec((D // 2,), lambda i: (0,)),
          pl.BlockSpec((D // 2,), lambda i: (0,)),
          pl.BlockSpec((1,), lambda i: (0,)),
      ],
      out_specs=pl.BlockSpec((BM,), lambda i: (i,)),
      out_shape=jax.ShapeDtypeStruct((N,), jnp.float32),
  )(agg2, cnt2, h1, W2l.T, b2l, W2r.T, Wf1.T, bf1, Wf2[0], bf2)

  return out


# speedup vs baseline: 7.0231x; 1.4636x over previous
import jax
import jax.numpy as jnp
from jax import lax
from jax.experimental import pallas as pl
from jax.experimental.pallas import tpu as pltpu
from jax.experimental.pallas import tpu_sc as plsc

N = 10000
E = 320000
D = 128
NP = 10240
NC, NS = 2, 16
NW = NC * NS
EPT = E // NW
C = 80
NCHUNK = EPT // C
RPT = NP // NS
BM = 1024


def _fill_iden(iden, s, lane):
  nj = C // 16

  def mkid(t, _):
    k = t // nj
    j = t % nj
    iden[k, pl.ds(j * 16, 16)] = s * RPT + k * C + j * 16 + lane
    return 0

  lax.fori_loop(0, (RPT // C) * nj, mkid, 0)


def _fill_const(buf, val):
  v = jnp.full((16,), val, jnp.float32)

  def fill(i, _):
    for j in range(D // 16):
      buf[i, pl.ds(j * 16, 16)] = v
    return 0

  lax.fori_loop(0, C, fill, 0)


def _sc_agg_body(x_hbm, src_hbm, dst_hbm, agg_hbm, sidx, didx, rows,
                 sidx2, didx2, rows2, iden, sema, semb, acc_sh):
  c = lax.axis_index("c")
  s = lax.axis_index("s")
  wid = s * NC + c
  ebase = wid * EPT

  lane = lax.iota(jnp.int32, 16)
  _fill_iden(iden, s, lane)
  _fill_const(rows, 0.0)

  for k in range(RPT // C):
    pltpu.sync_copy(rows, acc_sh.at[iden.at[k]])
  plsc.subcore_barrier()

  def load_gather(i, si, di, ro, sem):
    base = pl.multiple_of(ebase + i * C, 8)
    pltpu.sync_copy(src_hbm.at[pl.ds(base, C)], si)
    pltpu.sync_copy(dst_hbm.at[pl.ds(base, C)], di)
    return pltpu.async_copy(x_hbm.at[si], ro, sem)

  load_gather(0, sidx, didx, rows, sema)

  def pair(p, _):
    i0 = 2 * p
    load_gather(i0 + 1, sidx2, didx2, rows2, semb)
    pltpu.make_async_copy(x_hbm.at[sidx], rows, sema).wait()
    pltpu.sync_copy(rows, acc_sh.at[didx], add=True)
    load_gather(i0 + 2, sidx, didx, rows, sema)
    pltpu.make_async_copy(x_hbm.at[sidx2], rows2, semb).wait()
    pltpu.sync_copy(rows2, acc_sh.at[didx2], add=True)
    return 0

  lax.fori_loop(0, NCHUNK // 2, pair, 0)
  pltpu.make_async_copy(x_hbm.at[sidx], rows, sema).wait()
  pltpu.sync_copy(rows, acc_sh.at[didx], add=True)
  plsc.subcore_barrier()

  for k in range(RPT // C):
    off = pl.multiple_of(s * RPT + k * C, 8)
    pltpu.sync_copy(acc_sh.at[iden.at[k]], rows)
    pltpu.sync_copy(rows, agg_hbm.at[c, pl.ds(off, C)])


_sc_agg = pl.kernel(
    _sc_agg_body,
    out_type=jax.ShapeDtypeStruct((NC, NP, D), jnp.float32),
    mesh=plsc.VectorSubcoreMesh(core_axis_name="c", subcore_axis_name="s"),
    scratch_types=[
        pltpu.VMEM((C,), jnp.int32),
        pltpu.VMEM((C,), jnp.int32),
        pltpu.VMEM((C, D), jnp.float32),
        pltpu.VMEM((C,), jnp.int32),
        pltpu.VMEM((C,), jnp.int32),
        pltpu.VMEM((C, D), jnp.float32),
        pltpu.VMEM((RPT // C, C), jnp.int32),
        pltpu.SemaphoreType.DMA,
        pltpu.SemaphoreType.DMA,
        pltpu.VMEM_SHARED((NP, D), jnp.float32),
    ],
)


def _sc_cnt_body(dst_hbm, cnt_hbm, didx, ones, zero, iden, cnt_sh):
  c = lax.axis_index("c")
  s = lax.axis_index("s")
  wid = s * NC + c

  lane = lax.iota(jnp.int32, 16)
  _fill_iden(iden, s, lane)
  _fill_const(ones, 1.0)
  _fill_const(zero, 0.0)

  for k in range(RPT // C):
    pltpu.sync_copy(zero, cnt_sh.at[iden.at[k]])
  plsc.subcore_barrier()

  def step(i, _):
    base = pl.multiple_of(wid * EPT + i * C, 8)
    pltpu.sync_copy(dst_hbm.at[pl.ds(base, C)], didx)
    pltpu.sync_copy(ones, cnt_sh.at[didx], add=True)
    return 0

  lax.fori_loop(0, NCHUNK, step, 0)
  plsc.subcore_barrier()

  for k in range(RPT // C):
    off = pl.multiple_of(s * RPT + k * C, 8)
    pltpu.sync_copy(cnt_sh.at[iden.at[k]], zero)
    pltpu.sync_copy(zero, cnt_hbm.at[c, pl.ds(off, C)])


_sc_cnt = pl.kernel(
    _sc_cnt_body,
    out_type=jax.ShapeDtypeStruct((NC, NP, D), jnp.float32),
    mesh=plsc.VectorSubcoreMesh(core_axis_name="c", subcore_axis_name="s"),
    scratch_types=[
        pltpu.VMEM((C,), jnp.int32),
        pltpu.VMEM((C, D), jnp.float32),
        pltpu.VMEM((C, D), jnp.float32),
        pltpu.VMEM((RPT // C, C), jnp.int32),
        pltpu.VMEM_SHARED((NP, D), jnp.float32),
    ],
)


def _elu(h):
  return jnp.where(h > 0, h, jnp.exp(jnp.minimum(h, 0.0)) - 1.0)


def _tc1_body(agg_ref, cnt_ref, x_ref, wl_ref, bl_ref, wr_ref, o_ref):
  cnt = jnp.maximum(cnt_ref[:, 0] + cnt_ref[:, 1], 1.0)
  mean = (agg_ref[0] + agg_ref[1]) * (1.0 / cnt)[:, None]
  h = (jnp.dot(mean, wl_ref[...], preferred_element_type=jnp.float32)
       + jnp.dot(x_ref[...], wr_ref[...], preferred_element_type=jnp.float32)
       + bl_ref[...][None, :])
  o_ref[...] = _elu(h)


def _tc2_body(agg_ref, cnt_ref, h_ref, wl_ref, bl_ref, wr_ref,
              wf1_ref, bf1_ref, wf2_ref, bf2_ref, o_ref):
  cnt = jnp.maximum(cnt_ref[:, 0] + cnt_ref[:, 1], 1.0)
  mean = (agg_ref[0] + agg_ref[1]) * (1.0 / cnt)[:, None]
  h = (jnp.dot(mean, wl_ref[...], preferred_element_type=jnp.float32)
       + jnp.dot(h_ref[...], wr_ref[...], preferred_element_type=jnp.float32)
       + bl_ref[...][None, :])
  h = _elu(h)
  g = jnp.maximum(
      jnp.dot(h, wf1_ref[...], preferred_element_type=jnp.float32)
      + bf1_ref[...][None, :], 0.0)
  o_ref[...] = jnp.sum(g * wf2_ref[...][None, :], axis=1) + bf2_ref[...]


def kernel(x, edge_index, W1l, b1l, W1r, W2l, b2l, W2r, Wf1, bf1, Wf2, bf2):
  src = edge_index[0]
  dst = edge_index[1]

  cntp = _sc_cnt(dst)
  agg1 = _sc_agg(x, src, dst)
  cnt2 = cntp[:, :N, 0].T

  grid = (pl.cdiv(N, BM),)
  h1 = pl.pallas_call(
      _tc1_body,
      grid=grid,
      in_specs=[
          pl.BlockSpec((2, BM, D), lambda i: (0, i, 0)),
          pl.BlockSpec((BM, 2), lambda i: (i, 0)),
          pl.BlockSpec((BM, D), lambda i: (i, 0)),
          pl.BlockSpec((D, D), lambda i: (0, 0)),
          pl.BlockSpec((D,), lambda i: (0,)),
          pl.BlockSpec((D, D), lambda i: (0, 0)),
      ],
      out_specs=pl.BlockSpec((BM, D), lambda i: (i, 0)),
      out_shape=jax.ShapeDtypeStruct((N, D), jnp.float32),
  )(agg1, cnt2, x, W1l.T, b1l, W1r.T)

  agg2 = _sc_agg(h1, src, dst)

  out = pl.pallas_call(
      _tc2_body,
      grid=grid,
      in_specs=[
          pl.BlockSpec((2, BM, D), lambda i: (0, i, 0)),
          pl.BlockSpec((BM, 2), lambda i: (i, 0)),
          pl.BlockSpec((BM, D), lambda i: (i, 0)),
          pl.BlockSpec((D, D), lambda i: (0, 0)),
          pl.BlockSpec((D,), lambda i: (0,)),
          pl.BlockSpec((D, D), lambda i: (0, 0)),
          pl.BlockSpec((D, D // 2), lambda i: (0, 0)),
          pl.BlockSpec((D // 2,), lambda i: (0,)),
          pl.BlockSpec((D // 2,), lambda i: (0,)),
          pl.BlockSpec((1,), lambda i: (0,)),
      ],
      out_specs=pl.BlockSpec((BM,), lambda i: (i,)),
      out_shape=jax.ShapeDtypeStruct((N,), jnp.float32),
  )(agg2, cnt2, h1, W2l.T, b2l, W2r.T, Wf1.T, bf1, Wf2[0], bf2)

  return out


# speedup vs baseline: 7.7391x; 1.1020x over previous
import jax
import jax.numpy as jnp
from jax import lax
from jax.experimental import pallas as pl
from jax.experimental.pallas import tpu as pltpu
from jax.experimental.pallas import tpu_sc as plsc

N = 10000
E = 320000
D = 128
NP = 10240
NC, NS = 2, 16
NW = NC * NS
EPT = E // NW
C = 80
NCHUNK = EPT // C
RPT = NP // NS
BM = 1024


def _fill_iden(iden, s, lane):
  nj = C // 16

  def mkid(t, _):
    k = t // nj
    j = t % nj
    iden[k, pl.ds(j * 16, 16)] = s * RPT + k * C + j * 16 + lane
    return 0

  lax.fori_loop(0, (RPT // C) * nj, mkid, 0)


def _fill_const(buf, val):
  v = jnp.full((16,), val, jnp.float32)

  def fill(i, _):
    for j in range(D // 16):
      buf[i, pl.ds(j * 16, 16)] = v
    return 0

  lax.fori_loop(0, C, fill, 0)


def _sc_agg_body(x_hbm, src_hbm, dst_hbm, agg_hbm, sidx, didx, rows,
                 sidx2, didx2, rows2, iden, sema, semb, acc_sh):
  c = lax.axis_index("c")
  s = lax.axis_index("s")
  wid = s * NC + c
  ebase = wid * EPT

  lane = lax.iota(jnp.int32, 16)
  _fill_iden(iden, s, lane)
  _fill_const(rows, 0.0)

  for k in range(RPT // C):
    pltpu.sync_copy(rows, acc_sh.at[iden.at[k]])
  plsc.subcore_barrier()

  def load_gather(i, si, di, ro, sem):
    base = pl.multiple_of(ebase + i * C, 8)
    pltpu.sync_copy(src_hbm.at[pl.ds(base, C)], si)
    pltpu.sync_copy(dst_hbm.at[pl.ds(base, C)], di)
    return pltpu.async_copy(x_hbm.at[si], ro, sem)

  load_gather(0, sidx, didx, rows, sema)

  def pair(p, _):
    i0 = 2 * p
    load_gather(i0 + 1, sidx2, didx2, rows2, semb)
    pltpu.make_async_copy(x_hbm.at[sidx], rows, sema).wait()
    pltpu.sync_copy(rows, acc_sh.at[didx], add=True)
    load_gather(i0 + 2, sidx, didx, rows, sema)
    pltpu.make_async_copy(x_hbm.at[sidx2], rows2, semb).wait()
    pltpu.sync_copy(rows2, acc_sh.at[didx2], add=True)
    return 0

  lax.fori_loop(0, NCHUNK // 2, pair, 0)
  pltpu.make_async_copy(x_hbm.at[sidx], rows, sema).wait()
  pltpu.sync_copy(rows, acc_sh.at[didx], add=True)
  plsc.subcore_barrier()

  for k in range(RPT // C):
    off = pl.multiple_of(s * RPT + k * C, 8)
    pltpu.sync_copy(acc_sh.at[iden.at[k]], rows)
    pltpu.sync_copy(rows, agg_hbm.at[c, pl.ds(off, C)])


_sc_agg = pl.kernel(
    _sc_agg_body,
    out_type=jax.ShapeDtypeStruct((NC, NP, D), jnp.float32),
    mesh=plsc.VectorSubcoreMesh(core_axis_name="c", subcore_axis_name="s"),
    scratch_types=[
        pltpu.VMEM((C,), jnp.int32),
        pltpu.VMEM((C,), jnp.int32),
        pltpu.VMEM((C, D), jnp.float32),
        pltpu.VMEM((C,), jnp.int32),
        pltpu.VMEM((C,), jnp.int32),
        pltpu.VMEM((C, D), jnp.float32),
        pltpu.VMEM((RPT // C, C), jnp.int32),
        pltpu.SemaphoreType.DMA,
        pltpu.SemaphoreType.DMA,
        pltpu.VMEM_SHARED((NP, D), jnp.float32),
    ],
)


def _sc_cnt_body(dst_hbm, cnt_hbm, didx, didx2, ones, zero, iden,
                 sema, semb, cnt_sh):
  c = lax.axis_index("c")
  s = lax.axis_index("s")
  wid = s * NC + c
  ebase = wid * EPT

  lane = lax.iota(jnp.int32, 16)
  _fill_iden(iden, s, lane)
  _fill_const(ones, 1.0)
  _fill_const(zero, 0.0)

  for k in range(RPT // C):
    pltpu.sync_copy(zero, cnt_sh.at[iden.at[k]])
  plsc.subcore_barrier()

  def load(i, di):
    base = pl.multiple_of(ebase + i * C, 8)
    pltpu.sync_copy(dst_hbm.at[pl.ds(base, C)], di)

  load(0, didx)
  pltpu.async_copy(ones, cnt_sh.at[didx], sema, add=True)

  def pair(p, _):
    i0 = 2 * p
    load(i0 + 1, didx2)
    pltpu.async_copy(ones, cnt_sh.at[didx2], semb, add=True)
    pltpu.make_async_copy(ones, cnt_sh.at[didx], sema).wait()
    load(i0 + 2, didx)
    pltpu.async_copy(ones, cnt_sh.at[didx], sema, add=True)
    pltpu.make_async_copy(ones, cnt_sh.at[didx2], semb).wait()
    return 0

  lax.fori_loop(0, NCHUNK // 2, pair, 0)
  pltpu.make_async_copy(ones, cnt_sh.at[didx], sema).wait()
  plsc.subcore_barrier()

  for k in range(RPT // C):
    off = pl.multiple_of(s * RPT + k * C, 8)
    pltpu.sync_copy(cnt_sh.at[iden.at[k]], zero)
    pltpu.sync_copy(zero, cnt_hbm.at[c, pl.ds(off, C)])


_sc_cnt = pl.kernel(
    _sc_cnt_body,
    out_type=jax.ShapeDtypeStruct((NC, NP, D), jnp.float32),
    mesh=plsc.VectorSubcoreMesh(core_axis_name="c", subcore_axis_name="s"),
    scratch_types=[
        pltpu.VMEM((C,), jnp.int32),
        pltpu.VMEM((C,), jnp.int32),
        pltpu.VMEM((C, D), jnp.float32),
        pltpu.VMEM((C, D), jnp.float32),
        pltpu.VMEM((RPT // C, C), jnp.int32),
        pltpu.SemaphoreType.DMA,
        pltpu.SemaphoreType.DMA,
        pltpu.VMEM_SHARED((NP, D), jnp.float32),
    ],
)


def _elu(h):
  return jnp.where(h > 0, h, jnp.exp(jnp.minimum(h, 0.0)) - 1.0)


def _tc1_body(agg_ref, cnt_ref, x_ref, wl_ref, bl_ref, wr_ref, o_ref):
  cnt = jnp.maximum(cnt_ref[:, 0] + cnt_ref[:, 1], 1.0)
  mean = (agg_ref[0] + agg_ref[1]) * (1.0 / cnt)[:, None]
  h = (jnp.dot(mean, wl_ref[...], preferred_element_type=jnp.float32)
       + jnp.dot(x_ref[...], wr_ref[...], preferred_element_type=jnp.float32)
       + bl_ref[...][None, :])
  o_ref[...] = _elu(h)


def _tc2_body(agg_ref, cnt_ref, h_ref, wl_ref, bl_ref, wr_ref,
              wf1_ref, bf1_ref, wf2_ref, bf2_ref, o_ref):
  cnt = jnp.maximum(cnt_ref[:, 0] + cnt_ref[:, 1], 1.0)
  mean = (agg_ref[0] + agg_ref[1]) * (1.0 / cnt)[:, None]
  h = (jnp.dot(mean, wl_ref[...], preferred_element_type=jnp.float32)
       + jnp.dot(h_ref[...], wr_ref[...], preferred_element_type=jnp.float32)
       + bl_ref[...][None, :])
  h = _elu(h)
  g = jnp.maximum(
      jnp.dot(h, wf1_ref[...], preferred_element_type=jnp.float32)
      + bf1_ref[...][None, :], 0.0)
  o_ref[...] = jnp.sum(g * wf2_ref[...][None, :], axis=1) + bf2_ref[...]


def kernel(x, edge_index, W1l, b1l, W1r, W2l, b2l, W2r, Wf1, bf1, Wf2, bf2):
  src = edge_index[0]
  dst = edge_index[1]

  cntp = _sc_cnt(dst)
  agg1 = _sc_agg(x, src, dst)
  cnt2 = cntp[:, :N, 0].T

  grid = (pl.cdiv(N, BM),)
  h1 = pl.pallas_call(
      _tc1_body,
      grid=grid,
      in_specs=[
          pl.BlockSpec((2, BM, D), lambda i: (0, i, 0)),
          pl.BlockSpec((BM, 2), lambda i: (i, 0)),
          pl.BlockSpec((BM, D), lambda i: (i, 0)),
          pl.BlockSpec((D, D), lambda i: (0, 0)),
          pl.BlockSpec((D,), lambda i: (0,)),
          pl.BlockSpec((D, D), lambda i: (0, 0)),
      ],
      out_specs=pl.BlockSpec((BM, D), lambda i: (i, 0)),
      out_shape=jax.ShapeDtypeStruct((N, D), jnp.float32),
  )(agg1, cnt2, x, W1l.T, b1l, W1r.T)

  agg2 = _sc_agg(h1, src, dst)

  out = pl.pallas_call(
      _tc2_body,
      grid=grid,
      in_specs=[
          pl.BlockSpec((2, BM, D), lambda i: (0, i, 0)),
          pl.BlockSpec((BM, 2), lambda i: (i, 0)),
          pl.BlockSpec((BM, D), lambda i: (i, 0)),
          pl.BlockSpec((D, D), lambda i: (0, 0)),
          pl.BlockSpec((D,), lambda i: (0,)),
          pl.BlockSpec((D, D), lambda i: (0, 0)),
          pl.BlockSpec((D, D // 2), lambda i: (0, 0)),
          pl.BlockSpec((D // 2,), lambda i: (0,)),
          pl.BlockSpec((D // 2,), lambda i: (0,)),
          pl.BlockSpec((1,), lambda i: (0,)),
      ],
      out_specs=pl.BlockSpec((BM,), lambda i: (i,)),
      out_shape=jax.ShapeDtypeStruct((N,), jnp.float32),
  )(agg2, cnt2, h1, W2l.T, b2l, W2r.T, Wf1.T, bf1, Wf2[0], bf2)

  return out
